# Initial kernel scaffold; baseline (speedup 1.0000x reference)
import jax
import jax.numpy as jnp
from jax.experimental import pallas as pl


def kernel(x, W0, b0, W1, b1, Wmu, bmu, prototypes):
    raise NotImplementedError("write your pallas kernel here")


# speedup vs baseline: 6.6024x; 6.6024x over previous
import functools

import jax
import jax.numpy as jnp
from jax import lax
from jax.experimental import pallas as pl
from jax.experimental.pallas import tpu as pltpu
from jax.experimental.pallas import tpu_sc as plsc

_B = 16384
_IN = 256
_HID = 64
_OUT = 64
_K = 8192
_BETA = 0.25

_R1 = 1024
_R2 = 256

_DN = (((1,), (1,)), ((), ()))


def _mlp_body(x_ref, w0_ref, b0_ref, w1_ref, b1_ref, wmu_ref, bmu_ref, lat_ref):
    h = jax.nn.relu(lax.dot_general(x_ref[...], w0_ref[...], _DN) + b0_ref[...])
    h = jax.nn.relu(lax.dot_general(h, w1_ref[...], _DN) + b1_ref[...])
    lat_ref[...] = lax.dot_general(h, wmu_ref[...], _DN) + bmu_ref[...]


def _mlp(x, W0, b0, W1, b1, Wmu, bmu):
    return pl.pallas_call(
        _mlp_body,
        grid=(_B // _R1,),
        in_specs=[
            pl.BlockSpec((_R1, _IN), lambda i: (i, 0)),
            pl.BlockSpec((_HID, _IN), lambda i: (0, 0)),
            pl.BlockSpec((1, _HID), lambda i: (0, 0)),
            pl.BlockSpec((_OUT, _HID), lambda i: (0, 0)),
            pl.BlockSpec((1, _OUT), lambda i: (0, 0)),
            pl.BlockSpec((_OUT, _OUT), lambda i: (0, 0)),
            pl.BlockSpec((1, _OUT), lambda i: (0, 0)),
        ],
        out_specs=pl.BlockSpec((_R1, _OUT), lambda i: (i, 0)),
        out_shape=jax.ShapeDtypeStruct((_B, _OUT), jnp.float32),
    )(x, W0, b0, W1, b1, Wmu, bmu)


def _argmin_body(lat_ref, a_ref, b_ref, p_ref, idx_ref, minsum_ref):
    i = pl.program_id(0)
    c = lax.dot_general(lat_ref[...], p_ref[...], _DN)
    dists = (a_ref[...] + b_ref[...]) - 2.0 * c
    m = jnp.min(dists, axis=1, keepdims=True)
    iota = lax.broadcasted_iota(jnp.int32, dists.shape, 1)
    idx_ref[...] = jnp.min(jnp.where(dists == m, iota, _K), axis=1, keepdims=True)
    part = jnp.sum(m, axis=0, keepdims=True)

    @pl.when(i == 0)
    def _init():
        minsum_ref[...] = part

    @pl.when(i > 0)
    def _acc():
        minsum_ref[...] += part


def _argmin(latents, a, b, prototypes):
    return pl.pallas_call(
        _argmin_body,
        grid=(_B // _R2,),
        in_specs=[
            pl.BlockSpec((_R2, _OUT), lambda i: (i, 0)),
            pl.BlockSpec((_R2, 1), lambda i: (i, 0)),
            pl.BlockSpec((1, _K), lambda i: (0, 0)),
            pl.BlockSpec((_K, _OUT), lambda i: (0, 0)),
        ],
        out_specs=[
            pl.BlockSpec((_R2, 1), lambda i: (i, 0)),
            pl.BlockSpec((1, 1), lambda i: (0, 0)),
        ],
        out_shape=[
            jax.ShapeDtypeStruct((_B, 1), jnp.int32),
            jax.ShapeDtypeStruct((1, 1), jnp.float32),
        ],
    )(latents, a, b, prototypes)


_NC = 2
_NS = 16
_NW = _NC * _NS
_BPW = _B // _NW
_CH = 128
_GD = 128


def _gather_body(table_hbm, idx_hbm, out_hbm, idx_v, rows_v, sem):
    wid = lax.axis_index("s") * _NC + lax.axis_index("c")
    base = wid * _BPW
    pltpu.sync_copy(idx_hbm.at[pl.ds(base, _BPW)], idx_v)
    copies = [
        pltpu.async_copy(
            table_hbm.at[idx_v.at[pl.ds(j * _CH, _CH)]],
            rows_v.at[pl.ds(j * _CH, _CH), :],
            sem,
        )
        for j in range(_BPW // _CH)
    ]
    for cp in copies:
        cp.wait()
    pltpu.sync_copy(rows_v, out_hbm.at[pl.ds(base, _BPW)])


def _gather(table_padded, closest):
    mesh = plsc.VectorSubcoreMesh(core_axis_name="c", subcore_axis_name="s")
    run = functools.partial(
        pl.kernel,
        out_type=jax.ShapeDtypeStruct((_B, _GD), jnp.float32),
        mesh=mesh,
        scratch_types=[
            pltpu.VMEM((_BPW,), jnp.int32),
            pltpu.VMEM((_BPW, _GD), jnp.float32),
            pltpu.SemaphoreType.DMA,
        ],
    )(_gather_body)
    return run(table_padded, closest)


def kernel(x, W0, b0, W1, b1, Wmu, bmu, prototypes):
    latents = _mlp(x, W0, b0.reshape(1, _HID), W1, b1.reshape(1, _HID),
                   Wmu, bmu.reshape(1, _OUT))
    a = jnp.sum(latents ** 2, axis=1, keepdims=True)
    b = jnp.sum(prototypes ** 2, axis=1).reshape(1, _K)
    closest2d, minsum = _argmin(latents, a, b, prototypes)
    table_padded = jnp.pad(prototypes, ((0, 0), (0, _GD - _OUT)))
    quantized = _gather(table_padded, closest2d.reshape(_B))[:, :_OUT]
    m = minsum[0, 0] / jnp.float32(_B * _OUT)
    vq_loss = m * _BETA + m
    quantized_st = latents + (quantized - latents)
    capacity = jnp.asarray(0.0, dtype=jnp.float32)
    return quantized_st, vq_loss, capacity
